# baseline (speedup 1.0000x reference)
import functools

import jax
import jax.numpy as jnp
from jax import lax
from jax.experimental import pallas as pl
from jax.experimental.pallas import tpu as pltpu
from jax.experimental.pallas import tpu_sc as plsc

_NC, _NS = 2, 16
_NW = _NC * _NS
_W = 512


def _make_gather(n, d, dtype):
    per_w = n // _NW
    n_chunks = per_w // _W
    mesh = plsc.VectorSubcoreMesh(core_axis_name="c", subcore_axis_name="s")

    @functools.partial(
        pl.kernel,
        mesh=mesh,
        out_type=jax.ShapeDtypeStruct((n, d), dtype),
        scratch_types=[
            pltpu.VMEM((_W,), jnp.int32),
            pltpu.VMEM((_W, d), dtype),
            pltpu.SemaphoreType.DMA,
        ],
        compiler_params=pltpu.CompilerParams(use_tc_tiling_on_sc=False),
    )
    def k(table_hbm, idx_hbm, out_hbm, idx_v, rows_v, sem):
        wid = lax.axis_index("s") * _NC + lax.axis_index("c")
        base = wid * per_w

        @pl.loop(0, n_chunks)
        def _(c):
            off = base + c * _W
            pltpu.sync_copy(idx_hbm.at[pl.ds(off, _W)], idx_v)
            pltpu.async_copy(table_hbm.at[idx_v], rows_v, sem).wait()
            pltpu.sync_copy(rows_v, out_hbm.at[pl.ds(off, _W)])

    return k


def kernel(word_inp, table):
    b, l = word_inp.shape
    n = b * l
    d = table.shape[1]
    idx = word_inp.reshape(n)
    gathered = _make_gather(n, d, table.dtype)(table, idx)
    return gathered.reshape(b, l, d)


# speedup vs baseline: 1.0400x; 1.0400x over previous
import functools

import jax
import jax.numpy as jnp
from jax import lax
from jax.experimental import pallas as pl
from jax.experimental.pallas import tpu as pltpu
from jax.experimental.pallas import tpu_sc as plsc

_NC, _NS = 2, 16
_NW = _NC * _NS
_W = 640


def _make_gather(n, d, dtype):
    per_w = n // _NW
    n_chunks = per_w // _W
    assert n_chunks % 2 == 0 and n_chunks >= 4
    mesh = plsc.VectorSubcoreMesh(core_axis_name="c", subcore_axis_name="s")

    @functools.partial(
        pl.kernel,
        mesh=mesh,
        out_type=jax.ShapeDtypeStruct((n, d), dtype),
        scratch_types=[
            pltpu.VMEM((per_w,), jnp.int32),
            pltpu.VMEM((_W, d), dtype),
            pltpu.VMEM((_W, d), dtype),
            pltpu.SemaphoreType.DMA,
            pltpu.SemaphoreType.DMA,
            pltpu.SemaphoreType.DMA,
            pltpu.SemaphoreType.DMA,
        ],
        compiler_params=pltpu.CompilerParams(use_tc_tiling_on_sc=False),
    )
    def k(table_hbm, idx_hbm, out_hbm, idx_v, rows0, rows1, g0, g1, w0, w1):
        wid = lax.axis_index("s") * _NC + lax.axis_index("c")
        base = wid * per_w

        def start_gather(c, buf, sem):
            pltpu.make_async_copy(
                table_hbm.at[idx_v.at[pl.ds(c * _W, _W)]], buf, sem
            ).start()

        def wait_gather(c, buf, sem):
            pltpu.make_async_copy(
                table_hbm.at[idx_v.at[pl.ds(c * _W, _W)]], buf, sem
            ).wait()

        def start_write(c, buf, sem):
            pltpu.make_async_copy(
                buf, out_hbm.at[pl.ds(base + c * _W, _W)], sem
            ).start()

        def wait_write(c, buf, sem):
            pltpu.make_async_copy(
                buf, out_hbm.at[pl.ds(base + c * _W, _W)], sem
            ).wait()

        pltpu.sync_copy(idx_hbm.at[pl.ds(base, per_w)], idx_v)

        start_gather(0, rows0, g0)
        start_gather(1, rows1, g1)

        @pl.loop(0, n_chunks - 2, step=2)
        def _(c):
            wait_gather(c, rows0, g0)
            start_write(c, rows0, w0)
            wait_gather(c + 1, rows1, g1)
            start_write(c + 1, rows1, w1)
            wait_write(c, rows0, w0)
            start_gather(c + 2, rows0, g0)
            wait_write(c + 1, rows1, w1)
            start_gather(c + 3, rows1, g1)

        cl = n_chunks - 2
        wait_gather(cl, rows0, g0)
        start_write(cl, rows0, w0)
        wait_gather(cl + 1, rows1, g1)
        start_write(cl + 1, rows1, w1)
        wait_write(cl, rows0, w0)
        wait_write(cl + 1, rows1, w1)

    return k


def kernel(word_inp, table):
    b, l = word_inp.shape
    n = b * l
    d = table.shape[1]
    idx = word_inp.reshape(n)
    gathered = _make_gather(n, d, table.dtype)(table, idx)
    return gathered.reshape(b, l, d)


# speedup vs baseline: 1.2680x; 1.2192x over previous
import functools

import jax
import jax.numpy as jnp
from jax import lax
from jax.experimental import pallas as pl
from jax.experimental.pallas import tpu as pltpu
from jax.experimental.pallas import tpu_sc as plsc

_NC, _NS = 2, 16
_NW = _NC * _NS
_W = 256


def _make_gather(n, d, dp, dtype):
    per_w = n // _NW
    n_chunks = per_w // _W
    assert n_chunks % 2 == 0 and n_chunks >= 4
    mesh = plsc.VectorSubcoreMesh(core_axis_name="c", subcore_axis_name="s")

    @functools.partial(
        pl.kernel,
        mesh=mesh,
        out_type=jax.ShapeDtypeStruct((n, dp), dtype),
        scratch_types=[
            pltpu.VMEM((per_w,), jnp.int32),
            pltpu.VMEM((_W, dp), dtype),
            pltpu.VMEM((_W, dp), dtype),
            pltpu.SemaphoreType.DMA,
            pltpu.SemaphoreType.DMA,
            pltpu.SemaphoreType.DMA,
            pltpu.SemaphoreType.DMA,
        ],
    )
    def k(table_hbm, idx_hbm, out_hbm, idx_v, rows0, rows1, g0, g1, w0, w1):
        wid = lax.axis_index("s") * _NC + lax.axis_index("c")
        base = wid * per_w

        def start_gather(c, buf, sem):
            pltpu.make_async_copy(
                table_hbm.at[idx_v.at[pl.ds(c * _W, _W)]], buf, sem
            ).start()

        def wait_gather(c, buf, sem):
            pltpu.make_async_copy(
                table_hbm.at[idx_v.at[pl.ds(c * _W, _W)]], buf, sem
            ).wait()

        def start_write(c, buf, sem):
            pltpu.make_async_copy(
                buf, out_hbm.at[pl.ds(base + c * _W, _W)], sem
            ).start()

        def wait_write(c, buf, sem):
            pltpu.make_async_copy(
                buf, out_hbm.at[pl.ds(base + c * _W, _W)], sem
            ).wait()

        pltpu.sync_copy(idx_hbm.at[pl.ds(base, per_w)], idx_v)

        start_gather(0, rows0, g0)
        start_gather(1, rows1, g1)

        @pl.loop(0, n_chunks - 2, step=2)
        def _(c):
            wait_gather(c, rows0, g0)
            start_write(c, rows0, w0)
            wait_gather(c + 1, rows1, g1)
            start_write(c + 1, rows1, w1)
            wait_write(c, rows0, w0)
            start_gather(c + 2, rows0, g0)
            wait_write(c + 1, rows1, w1)
            start_gather(c + 3, rows1, g1)

        cl = n_chunks - 2
        wait_gather(cl, rows0, g0)
        start_write(cl, rows0, w0)
        wait_gather(cl + 1, rows1, g1)
        start_write(cl + 1, rows1, w1)
        wait_write(cl, rows0, w0)
        wait_write(cl + 1, rows1, w1)

    return k


def kernel(word_inp, table):
    b, l = word_inp.shape
    n = b * l
    d = table.shape[1]
    dp = 2 * d
    idx = word_inp.reshape(n)
    table_p = jnp.pad(table, ((0, 0), (0, dp - d)))
    gathered = _make_gather(n, d, dp, table.dtype)(table_p, idx)
    return gathered[:, :d].reshape(b, l, d)
